# baseline (speedup 1.0000x reference)
import functools

import jax
import jax.numpy as jnp
from jax import lax
from jax.experimental import pallas as pl
from jax.experimental.pallas import tpu as pltpu
from jax.experimental.pallas import tpu_sc as plsc

S = 2048
H = 2048
E = 8
FF = 2048
BM = 64
NP = S * 2 + E * BM
NB = NP // BM
NBPAD = ((NB + 15) // 16) * 16

RT = 256

_MESH = plsc.VectorSubcoreMesh(core_axis_name="c", subcore_axis_name="s")
_SC_PARAMS = pltpu.CompilerParams(needs_layout_passes=False)
NW = 32
TPW = S // NW

_ONES16 = None


def _splat(vec, i):
    tt = jnp.full((16, 1), i, jnp.int32)
    dnums = lax.GatherDimensionNumbers(
        offset_dims=(), collapsed_slice_dims=(0,), start_index_map=(0,))
    return lax.gather(vec, tt, dnums, (1,),
                      mode=lax.GatherScatterMode.PROMISE_IN_BOUNDS)


def _router_body(x_ref, gate_ref, selw_ref, seli_ref):
    x = x_ref[...]
    g = gate_ref[...]
    lt = lax.dot_general(g, x, (((1,), (1,)), ((), ())),
                         preferred_element_type=jnp.float32)
    row = lax.broadcasted_iota(jnp.int32, lt.shape, 0)
    m = jnp.max(lt, axis=0, keepdims=True)
    p = jnp.exp(lt - m)
    p = p / jnp.sum(p, axis=0, keepdims=True)
    m1 = jnp.max(p, axis=0, keepdims=True)
    a1 = jnp.min(jnp.where(p == m1, row, E), axis=0, keepdims=True)
    p2 = jnp.where(row == a1, -1.0, p)
    m2 = jnp.max(p2, axis=0, keepdims=True)
    a2 = jnp.min(jnp.where(p2 == m2, row, E), axis=0, keepdims=True)
    tot = m1 + m2
    selw_ref[...] = jnp.where(row == 0, m1 / tot,
                              jnp.where(row == 1, m2 / tot, 0.0))
    seli_ref[...] = jnp.where(row == 0, a1, jnp.where(row == 1, a2, 0))


def _router(x, gate_w):
    return pl.pallas_call(
        _router_body,
        grid=(S // RT,),
        in_specs=[pl.BlockSpec((RT, H), lambda i: (i, 0)),
                  pl.BlockSpec((E, H), lambda i: (0, 0))],
        out_specs=[pl.BlockSpec((E, RT), lambda i: (0, i)),
                   pl.BlockSpec((E, RT), lambda i: (0, i))],
        out_shape=[jax.ShapeDtypeStruct((E, S), jnp.float32),
                   jax.ShapeDtypeStruct((E, S), jnp.int32)],
    )(x, gate_w)


def _sort_body(x_hbm, si_hbm, xs_hbm, d0_hbm, d1_hbm, be_hbm,
               s0v, s1v, d0v, d1v, bev, cnt, cntr, xb, i0v, i1v, shrd, sem):
    cid = lax.axis_index("c")
    sid = lax.axis_index("s")
    wid = sid * 2 + cid
    ones = jnp.ones((16,), jnp.int32)
    lanes = lax.iota(jnp.int32, 16)

    @pl.when(sid == 0)
    def _():
        pltpu.async_copy(si_hbm.at[0], s0v, sem).wait()
        pltpu.async_copy(si_hbm.at[1], s1v, sem).wait()
        cnt[...] = jnp.zeros((16,), jnp.int32)

        @pl.loop(0, S // 16)
        def _(i):
            plsc.addupdate_scatter(cnt, [s0v[pl.ds(i * 16, 16)]], ones)
            plsc.addupdate_scatter(cnt, [s1v[pl.ds(i * 16, 16)]], ones)

        c = cnt[...]
        nblk = jnp.where(lanes < E, (c + BM - 1) // BM, 0)
        cs = plsc.cumsum(nblk)
        bstart = cs - nblk
        cntr[...] = bstart * BM

        for j in range(NBPAD // 16):
            v = lanes + 16 * j
            be = jnp.full((16,), -1, jnp.int32)
            for e in range(E):
                be = be + jnp.where(v >= _splat(bstart, e), 1, 0)
            bev[pl.ds(16 * j, 16)] = jnp.clip(be, 0, E - 1)

        def dest_pass(sv, dv):
            @pl.loop(0, S // 16)
            def _(i):
                v = sv[pl.ds(i * 16, 16)]
                rk, _last = plsc.scan_count(v)
                base = plsc.load_gather(cntr, [v])
                dv[pl.ds(i * 16, 16)] = base + rk - 1
                plsc.addupdate_scatter(cntr, [v], ones)

        dest_pass(s0v, d0v)
        dest_pass(s1v, d1v)

        pltpu.sync_copy(d0v, shrd.at[0])
        pltpu.sync_copy(d1v, shrd.at[1])

        @pl.when(cid == 0)
        def _():
            pltpu.async_copy(d0v, d0_hbm, sem).wait()
            pltpu.async_copy(d1v, d1_hbm, sem).wait()
            pltpu.async_copy(bev, be_hbm, sem).wait()

    plsc.subcore_barrier()

    base = wid * TPW
    pltpu.sync_copy(shrd.at[0, pl.ds(base, TPW)], i0v)
    pltpu.sync_copy(shrd.at[1, pl.ds(base, TPW)], i1v)
    for cch in range(TPW // 16):
        pltpu.sync_copy(x_hbm.at[pl.ds(base + cch * 16, 16)], xb)
        iv0 = i0v[pl.ds(cch * 16, 16)]
        iv1 = i1v[pl.ds(cch * 16, 16)]
        pltpu.sync_copy(xb, xs_hbm.at[iv0])
        pltpu.sync_copy(xb, xs_hbm.at[iv1])


_sort_kernel = functools.partial(
    pl.kernel,
    out_type=(jax.ShapeDtypeStruct((NP, H), jnp.float32),
              jax.ShapeDtypeStruct((S,), jnp.int32),
              jax.ShapeDtypeStruct((S,), jnp.int32),
              jax.ShapeDtypeStruct((NBPAD,), jnp.int32)),
    mesh=_MESH,
    scratch_types=[pltpu.VMEM((S,), jnp.int32), pltpu.VMEM((S,), jnp.int32),
                   pltpu.VMEM((S,), jnp.int32), pltpu.VMEM((S,), jnp.int32),
                   pltpu.VMEM((NBPAD,), jnp.int32),
                   pltpu.VMEM((16,), jnp.int32), pltpu.VMEM((16,), jnp.int32),
                   pltpu.VMEM((16, H), jnp.float32),
                   pltpu.VMEM((TPW,), jnp.int32), pltpu.VMEM((TPW,), jnp.int32),
                   pltpu.VMEM_SHARED((2, S), jnp.int32),
                   pltpu.SemaphoreType.DMA],
    compiler_params=_SC_PARAMS,
)(_sort_body)


def _ffn_body(be_ref, xs_ref, w1_ref, w3_ref, w2_ref, ys_ref):
    del be_ref
    x = xs_ref[...].astype(jnp.bfloat16)
    h1 = lax.dot_general(x, w1_ref[0], (((1,), (1,)), ((), ())),
                         preferred_element_type=jnp.float32)
    h3 = lax.dot_general(x, w3_ref[0], (((1,), (1,)), ((), ())),
                         preferred_element_type=jnp.float32)
    h = (h1 * jax.nn.sigmoid(h1)) * h3
    y = lax.dot_general(h.astype(jnp.bfloat16), w2_ref[0],
                        (((1,), (1,)), ((), ())),
                        preferred_element_type=jnp.float32)
    ys_ref[...] = y


def _ffn(blk_exp, xs, w1b, w3b, w2b):
    return pl.pallas_call(
        _ffn_body,
        grid_spec=pltpu.PrefetchScalarGridSpec(
            num_scalar_prefetch=1,
            grid=(NB,),
            in_specs=[pl.BlockSpec((BM, H), lambda i, be: (i, 0)),
                      pl.BlockSpec((1, FF, H), lambda i, be: (be[i], 0, 0)),
                      pl.BlockSpec((1, FF, H), lambda i, be: (be[i], 0, 0)),
                      pl.BlockSpec((1, H, FF), lambda i, be: (be[i], 0, 0))],
            out_specs=pl.BlockSpec((BM, H), lambda i, be: (i, 0)),
        ),
        out_shape=jax.ShapeDtypeStruct((NP, H), jnp.float32),
    )(blk_exp, xs, w1b, w3b, w2b)


def _combine_body(ys_hbm, sw_hbm, d0_hbm, d1_hbm, out_hbm,
                  i0v, i1v, w0v, w1v, g0, g1, ob, sem0, sem1):
    cid = lax.axis_index("c")
    sid = lax.axis_index("s")
    wid = sid * 2 + cid
    base = wid * TPW
    pltpu.sync_copy(d0_hbm.at[pl.ds(base, TPW)], i0v)
    pltpu.sync_copy(d1_hbm.at[pl.ds(base, TPW)], i1v)
    pltpu.sync_copy(sw_hbm.at[0, pl.ds(base, TPW)], w0v)
    pltpu.sync_copy(sw_hbm.at[1, pl.ds(base, TPW)], w1v)
    for cch in range(TPW // 16):
        iv0 = i0v[pl.ds(cch * 16, 16)]
        iv1 = i1v[pl.ds(cch * 16, 16)]
        cp0 = pltpu.async_copy(ys_hbm.at[iv0], g0, sem0)
        cp1 = pltpu.async_copy(ys_hbm.at[iv1], g1, sem1)
        cp0.wait()
        cp1.wait()
        wa = w0v[pl.ds(cch * 16, 16)]
        wb = w1v[pl.ds(cch * 16, 16)]

        @pl.loop(0, 16)
        def _(t):
            was = _splat(wa, t)
            wbs = _splat(wb, t)

            @pl.loop(0, 8)
            def _(u):
                for uu in range(16):
                    sl = pl.ds(u * 256 + uu * 16, 16)
                    ob[t, sl] = g0[t, sl] * was + g1[t, sl] * wbs

        pltpu.sync_copy(ob, out_hbm.at[pl.ds(base + cch * 16, 16)])


_combine_kernel = functools.partial(
    pl.kernel,
    out_type=jax.ShapeDtypeStruct((S, H), jnp.float32),
    mesh=_MESH,
    scratch_types=[pltpu.VMEM((TPW,), jnp.int32), pltpu.VMEM((TPW,), jnp.int32),
                   pltpu.VMEM((TPW,), jnp.float32),
                   pltpu.VMEM((TPW,), jnp.float32),
                   pltpu.VMEM((16, H), jnp.float32),
                   pltpu.VMEM((16, H), jnp.float32),
                   pltpu.VMEM((16, H), jnp.float32),
                   pltpu.SemaphoreType.DMA, pltpu.SemaphoreType.DMA],
    compiler_params=_SC_PARAMS,
)(_combine_body)


def kernel(hidden_states, gate_w, w1, w3, w2):
    x = hidden_states.reshape(S, H)
    w1b = w1.astype(jnp.bfloat16)
    w3b = w3.astype(jnp.bfloat16)
    w2b = w2.astype(jnp.bfloat16)

    selw, seli = _router(x, gate_w)

    xs, d0, d1, blk_exp = _sort_kernel(x, seli)

    ys = _ffn(blk_exp, xs, w1b, w3b, w2b)

    out = _combine_kernel(ys, selw, d0, d1)
    return out.reshape(hidden_states.shape)


# speedup vs baseline: 1.2978x; 1.2978x over previous
import functools

import jax
import jax.numpy as jnp
from jax import lax
from jax.experimental import pallas as pl
from jax.experimental.pallas import tpu as pltpu
from jax.experimental.pallas import tpu_sc as plsc

S = 2048
H = 2048
E = 8
FF = 2048
BM = 128
NP = S * 2 + E * BM
NB = NP // BM
NBPAD = ((NB + 15) // 16) * 16

RT = 256

_MESH = plsc.VectorSubcoreMesh(core_axis_name="c", subcore_axis_name="s")
_SC_PARAMS = pltpu.CompilerParams(needs_layout_passes=False)
NW = 32
TPW = S // NW

_ONES16 = None


def _splat(vec, i):
    tt = jnp.full((16, 1), i, jnp.int32)
    dnums = lax.GatherDimensionNumbers(
        offset_dims=(), collapsed_slice_dims=(0,), start_index_map=(0,))
    return lax.gather(vec, tt, dnums, (1,),
                      mode=lax.GatherScatterMode.PROMISE_IN_BOUNDS)


def _router_body(x_ref, gate_ref, selw_ref, seli_ref):
    x = x_ref[...]
    g = gate_ref[...]
    lt = lax.dot_general(g, x, (((1,), (1,)), ((), ())),
                         preferred_element_type=jnp.float32)
    row = lax.broadcasted_iota(jnp.int32, lt.shape, 0)
    m = jnp.max(lt, axis=0, keepdims=True)
    p = jnp.exp(lt - m)
    p = p / jnp.sum(p, axis=0, keepdims=True)
    m1 = jnp.max(p, axis=0, keepdims=True)
    a1 = jnp.min(jnp.where(p == m1, row, E), axis=0, keepdims=True)
    p2 = jnp.where(row == a1, -1.0, p)
    m2 = jnp.max(p2, axis=0, keepdims=True)
    a2 = jnp.min(jnp.where(p2 == m2, row, E), axis=0, keepdims=True)
    tot = m1 + m2
    selw_ref[...] = jnp.where(row == 0, m1 / tot,
                              jnp.where(row == 1, m2 / tot, 0.0))
    seli_ref[...] = jnp.where(row == 0, a1, jnp.where(row == 1, a2, 0))


def _router(x, gate_w):
    return pl.pallas_call(
        _router_body,
        grid=(S // RT,),
        in_specs=[pl.BlockSpec((RT, H), lambda i: (i, 0)),
                  pl.BlockSpec((E, H), lambda i: (0, 0))],
        out_specs=[pl.BlockSpec((E, RT), lambda i: (0, i)),
                   pl.BlockSpec((E, RT), lambda i: (0, i))],
        out_shape=[jax.ShapeDtypeStruct((E, S), jnp.float32),
                   jax.ShapeDtypeStruct((E, S), jnp.int32)],
    )(x, gate_w)


def _sort_body(x_hbm, si_hbm, xs_hbm, d0_hbm, d1_hbm, be_hbm,
               s0v, s1v, d0v, d1v, bev, cnt, cntr, xb, i0v, i1v, shrd, sem):
    cid = lax.axis_index("c")
    sid = lax.axis_index("s")
    wid = sid * 2 + cid
    ones = jnp.ones((16,), jnp.int32)
    lanes = lax.iota(jnp.int32, 16)

    @pl.when(sid == 0)
    def _():
        pltpu.async_copy(si_hbm.at[0], s0v, sem).wait()
        pltpu.async_copy(si_hbm.at[1], s1v, sem).wait()
        cnt[...] = jnp.zeros((16,), jnp.int32)

        @pl.loop(0, S // 16)
        def _(i):
            plsc.addupdate_scatter(cnt, [s0v[pl.ds(i * 16, 16)]], ones)
            plsc.addupdate_scatter(cnt, [s1v[pl.ds(i * 16, 16)]], ones)

        c = cnt[...]
        nblk = jnp.where(lanes < E, (c + BM - 1) // BM, 0)
        cs = plsc.cumsum(nblk)
        bstart = cs - nblk
        cntr[...] = bstart * BM

        for j in range(NBPAD // 16):
            v = lanes + 16 * j
            be = jnp.full((16,), -1, jnp.int32)
            for e in range(E):
                be = be + jnp.where(v >= _splat(bstart, e), 1, 0)
            bev[pl.ds(16 * j, 16)] = jnp.clip(be, 0, E - 1)

        def dest_pass(sv, dv):
            @pl.loop(0, S // 16)
            def _(i):
                v = sv[pl.ds(i * 16, 16)]
                rk, _last = plsc.scan_count(v)
                base = plsc.load_gather(cntr, [v])
                dv[pl.ds(i * 16, 16)] = base + rk - 1
                plsc.addupdate_scatter(cntr, [v], ones)

        dest_pass(s0v, d0v)
        dest_pass(s1v, d1v)

        pltpu.sync_copy(d0v, shrd.at[0])
        pltpu.sync_copy(d1v, shrd.at[1])

        @pl.when(cid == 0)
        def _():
            pltpu.async_copy(d0v, d0_hbm, sem).wait()
            pltpu.async_copy(d1v, d1_hbm, sem).wait()
            pltpu.async_copy(bev, be_hbm, sem).wait()

    plsc.subcore_barrier()

    base = wid * TPW
    pltpu.sync_copy(shrd.at[0, pl.ds(base, TPW)], i0v)
    pltpu.sync_copy(shrd.at[1, pl.ds(base, TPW)], i1v)
    for cch in range(TPW // 16):
        pltpu.sync_copy(x_hbm.at[pl.ds(base + cch * 16, 16)], xb)
        iv0 = i0v[pl.ds(cch * 16, 16)]
        iv1 = i1v[pl.ds(cch * 16, 16)]
        pltpu.sync_copy(xb, xs_hbm.at[iv0])
        pltpu.sync_copy(xb, xs_hbm.at[iv1])


_sort_kernel = functools.partial(
    pl.kernel,
    out_type=(jax.ShapeDtypeStruct((NP, H), jnp.float32),
              jax.ShapeDtypeStruct((S,), jnp.int32),
              jax.ShapeDtypeStruct((S,), jnp.int32),
              jax.ShapeDtypeStruct((NBPAD,), jnp.int32)),
    mesh=_MESH,
    scratch_types=[pltpu.VMEM((S,), jnp.int32), pltpu.VMEM((S,), jnp.int32),
                   pltpu.VMEM((S,), jnp.int32), pltpu.VMEM((S,), jnp.int32),
                   pltpu.VMEM((NBPAD,), jnp.int32),
                   pltpu.VMEM((16,), jnp.int32), pltpu.VMEM((16,), jnp.int32),
                   pltpu.VMEM((16, H), jnp.float32),
                   pltpu.VMEM((TPW,), jnp.int32), pltpu.VMEM((TPW,), jnp.int32),
                   pltpu.VMEM_SHARED((2, S), jnp.int32),
                   pltpu.SemaphoreType.DMA],
    compiler_params=_SC_PARAMS,
)(_sort_body)


def _ffn_body(be_ref, xs_ref, w1_ref, w3_ref, w2_ref, ys_ref):
    del be_ref
    x = xs_ref[...].astype(jnp.bfloat16)
    h1 = lax.dot_general(x, w1_ref[0], (((1,), (1,)), ((), ())),
                         preferred_element_type=jnp.float32)
    h3 = lax.dot_general(x, w3_ref[0], (((1,), (1,)), ((), ())),
                         preferred_element_type=jnp.float32)
    h = (h1 * jax.nn.sigmoid(h1)) * h3
    y = lax.dot_general(h.astype(jnp.bfloat16), w2_ref[0],
                        (((1,), (1,)), ((), ())),
                        preferred_element_type=jnp.float32)
    ys_ref[...] = y


def _ffn(blk_exp, xs, w1b, w3b, w2b):
    return pl.pallas_call(
        _ffn_body,
        grid_spec=pltpu.PrefetchScalarGridSpec(
            num_scalar_prefetch=1,
            grid=(NB,),
            in_specs=[pl.BlockSpec((BM, H), lambda i, be: (i, 0)),
                      pl.BlockSpec((1, FF, H), lambda i, be: (be[i], 0, 0)),
                      pl.BlockSpec((1, FF, H), lambda i, be: (be[i], 0, 0)),
                      pl.BlockSpec((1, H, FF), lambda i, be: (be[i], 0, 0))],
            out_specs=pl.BlockSpec((BM, H), lambda i, be: (i, 0)),
        ),
        out_shape=jax.ShapeDtypeStruct((NP, H), jnp.float32),
    )(blk_exp, xs, w1b, w3b, w2b)


def _combine_body(ys_hbm, sw_hbm, d0_hbm, d1_hbm, out_hbm,
                  i0v, i1v, w0v, w1v, g0, g1, ob, sem0, sem1):
    cid = lax.axis_index("c")
    sid = lax.axis_index("s")
    wid = sid * 2 + cid
    base = wid * TPW
    pltpu.sync_copy(d0_hbm.at[pl.ds(base, TPW)], i0v)
    pltpu.sync_copy(d1_hbm.at[pl.ds(base, TPW)], i1v)
    pltpu.sync_copy(sw_hbm.at[0, pl.ds(base, TPW)], w0v)
    pltpu.sync_copy(sw_hbm.at[1, pl.ds(base, TPW)], w1v)
    for cch in range(TPW // 16):
        iv0 = i0v[pl.ds(cch * 16, 16)]
        iv1 = i1v[pl.ds(cch * 16, 16)]
        cp0 = pltpu.async_copy(ys_hbm.at[iv0], g0, sem0)
        cp1 = pltpu.async_copy(ys_hbm.at[iv1], g1, sem1)
        cp0.wait()
        cp1.wait()
        wa = w0v[pl.ds(cch * 16, 16)]
        wb = w1v[pl.ds(cch * 16, 16)]

        @pl.loop(0, 16)
        def _(t):
            was = _splat(wa, t)
            wbs = _splat(wb, t)

            @pl.loop(0, 8)
            def _(u):
                for uu in range(16):
                    sl = pl.ds(u * 256 + uu * 16, 16)
                    ob[t, sl] = g0[t, sl] * was + g1[t, sl] * wbs

        pltpu.sync_copy(ob, out_hbm.at[pl.ds(base + cch * 16, 16)])


_combine_kernel = functools.partial(
    pl.kernel,
    out_type=jax.ShapeDtypeStruct((S, H), jnp.float32),
    mesh=_MESH,
    scratch_types=[pltpu.VMEM((TPW,), jnp.int32), pltpu.VMEM((TPW,), jnp.int32),
                   pltpu.VMEM((TPW,), jnp.float32),
                   pltpu.VMEM((TPW,), jnp.float32),
                   pltpu.VMEM((16, H), jnp.float32),
                   pltpu.VMEM((16, H), jnp.float32),
                   pltpu.VMEM((16, H), jnp.float32),
                   pltpu.SemaphoreType.DMA, pltpu.SemaphoreType.DMA],
    compiler_params=_SC_PARAMS,
)(_combine_body)


def kernel(hidden_states, gate_w, w1, w3, w2):
    x = hidden_states.reshape(S, H)
    w1b = w1.astype(jnp.bfloat16)
    w3b = w3.astype(jnp.bfloat16)
    w2b = w2.astype(jnp.bfloat16)

    selw, seli = _router(x, gate_w)

    xs, d0, d1, blk_exp = _sort_kernel(x, seli)

    ys = _ffn(blk_exp, xs, w1b, w3b, w2b)

    out = _combine_kernel(ys, selw, d0, d1)
    return out.reshape(hidden_states.shape)


# speedup vs baseline: 1.7405x; 1.3411x over previous
import functools

import jax
import jax.numpy as jnp
from jax import lax
from jax.experimental import pallas as pl
from jax.experimental.pallas import tpu as pltpu
from jax.experimental.pallas import tpu_sc as plsc

S = 2048
H = 2048
E = 8
FF = 2048
BM = 256
NP = S * 2 + E * BM
NB = NP // BM
NBPAD = ((NB + 15) // 16) * 16

RT = 256

_MESH = plsc.VectorSubcoreMesh(core_axis_name="c", subcore_axis_name="s")
_SC_PARAMS = pltpu.CompilerParams(needs_layout_passes=False)
NW = 32
TPW = S // NW

_ONES16 = None


def _splat(vec, i):
    tt = jnp.full((16, 1), i, jnp.int32)
    dnums = lax.GatherDimensionNumbers(
        offset_dims=(), collapsed_slice_dims=(0,), start_index_map=(0,))
    return lax.gather(vec, tt, dnums, (1,),
                      mode=lax.GatherScatterMode.PROMISE_IN_BOUNDS)


def _router_body(x_ref, gate_ref, selw_ref, seli_ref):
    x = x_ref[...]
    g = gate_ref[...]
    lt = lax.dot_general(g, x, (((1,), (1,)), ((), ())),
                         preferred_element_type=jnp.float32)
    row = lax.broadcasted_iota(jnp.int32, lt.shape, 0)
    m = jnp.max(lt, axis=0, keepdims=True)
    p = jnp.exp(lt - m)
    p = p / jnp.sum(p, axis=0, keepdims=True)
    m1 = jnp.max(p, axis=0, keepdims=True)
    a1 = jnp.min(jnp.where(p == m1, row, E), axis=0, keepdims=True)
    p2 = jnp.where(row == a1, -1.0, p)
    m2 = jnp.max(p2, axis=0, keepdims=True)
    a2 = jnp.min(jnp.where(p2 == m2, row, E), axis=0, keepdims=True)
    tot = m1 + m2
    selw_ref[...] = jnp.where(row == 0, m1 / tot,
                              jnp.where(row == 1, m2 / tot, 0.0))
    seli_ref[...] = jnp.where(row == 0, a1, jnp.where(row == 1, a2, 0))


def _router(x, gate_w):
    return pl.pallas_call(
        _router_body,
        grid=(S // RT,),
        in_specs=[pl.BlockSpec((RT, H), lambda i: (i, 0)),
                  pl.BlockSpec((E, H), lambda i: (0, 0))],
        out_specs=[pl.BlockSpec((E, RT), lambda i: (0, i)),
                   pl.BlockSpec((E, RT), lambda i: (0, i))],
        out_shape=[jax.ShapeDtypeStruct((E, S), jnp.float32),
                   jax.ShapeDtypeStruct((E, S), jnp.int32)],
    )(x, gate_w)


def _sort_body(x_hbm, si_hbm, xs_hbm, d0_hbm, d1_hbm, be_hbm,
               s0v, s1v, d0v, d1v, bev, cnt, cntr, xb0, xb1, i0v, i1v, shrd,
               sem, seml0, seml1, semsc0, semsc1):
    cid = lax.axis_index("c")
    sid = lax.axis_index("s")
    wid = sid * 2 + cid
    ones = jnp.ones((16,), jnp.int32)
    lanes = lax.iota(jnp.int32, 16)
    base = wid * TPW
    lds = [pltpu.async_copy(x_hbm.at[pl.ds(base, 16)], xb0, seml0),
           None, None, None]

    @pl.when(sid == 0)
    def _():
        pltpu.async_copy(si_hbm.at[0], s0v, sem).wait()
        pltpu.async_copy(si_hbm.at[1], s1v, sem).wait()
        cnt[...] = jnp.zeros((16,), jnp.int32)

        @pl.loop(0, S // 16)
        def _(i):
            plsc.addupdate_scatter(cnt, [s0v[pl.ds(i * 16, 16)]], ones)
            plsc.addupdate_scatter(cnt, [s1v[pl.ds(i * 16, 16)]], ones)

        c = cnt[...]
        nblk = jnp.where(lanes < E, (c + BM - 1) // BM, 0)
        cs = plsc.cumsum(nblk)
        bstart = cs - nblk
        cntr[...] = bstart * BM

        for j in range(NBPAD // 16):
            v = lanes + 16 * j
            be = jnp.full((16,), -1, jnp.int32)
            for e in range(E):
                be = be + jnp.where(v >= _splat(bstart, e), 1, 0)
            bev[pl.ds(16 * j, 16)] = jnp.clip(be, 0, E - 1)

        def dest_pass(sv, dv):
            @pl.loop(0, S // 16)
            def _(i):
                v = sv[pl.ds(i * 16, 16)]
                rk, _last = plsc.scan_count(v)
                base = plsc.load_gather(cntr, [v])
                dv[pl.ds(i * 16, 16)] = base + rk - 1
                plsc.addupdate_scatter(cntr, [v], ones)

        dest_pass(s0v, d0v)
        dest_pass(s1v, d1v)

        pltpu.sync_copy(d0v, shrd.at[0])
        pltpu.sync_copy(d1v, shrd.at[1])

        @pl.when(cid == 0)
        def _():
            pltpu.async_copy(d0v, d0_hbm, sem).wait()
            pltpu.async_copy(d1v, d1_hbm, sem).wait()
            pltpu.async_copy(bev, be_hbm, sem).wait()

    plsc.subcore_barrier()

    pltpu.sync_copy(shrd.at[0, pl.ds(base, TPW)], i0v)
    pltpu.sync_copy(shrd.at[1, pl.ds(base, TPW)], i1v)
    xbufs = [xb0, xb1]
    lsems = [seml0, seml1]
    ssems = [semsc0, semsc1]
    nch = TPW // 16
    lds[1] = pltpu.async_copy(x_hbm.at[pl.ds(base + 16, 16)], xb1, seml1)
    pend = [None, None]
    for cch in range(nch):
        p = cch % 2
        buf = xbufs[p]
        lds[cch].wait()
        iv0 = i0v[pl.ds(cch * 16, 16)]
        iv1 = i1v[pl.ds(cch * 16, 16)]
        s0 = pltpu.async_copy(buf, xs_hbm.at[iv0], ssems[p])
        s1 = pltpu.async_copy(buf, xs_hbm.at[iv1], ssems[p])
        if cch + 2 < nch:
            s0.wait()
            s1.wait()
            lds[cch + 2] = pltpu.async_copy(
                x_hbm.at[pl.ds(base + (cch + 2) * 16, 16)], buf, lsems[p])
        else:
            pend[p] = (s0, s1)
    for pr in pend:
        if pr is not None:
            pr[0].wait()
            pr[1].wait()


_sort_kernel = functools.partial(
    pl.kernel,
    out_type=(jax.ShapeDtypeStruct((NP, H), jnp.float32),
              jax.ShapeDtypeStruct((S,), jnp.int32),
              jax.ShapeDtypeStruct((S,), jnp.int32),
              jax.ShapeDtypeStruct((NBPAD,), jnp.int32)),
    mesh=_MESH,
    scratch_types=[pltpu.VMEM((S,), jnp.int32), pltpu.VMEM((S,), jnp.int32),
                   pltpu.VMEM((S,), jnp.int32), pltpu.VMEM((S,), jnp.int32),
                   pltpu.VMEM((NBPAD,), jnp.int32),
                   pltpu.VMEM((16,), jnp.int32), pltpu.VMEM((16,), jnp.int32),
                   pltpu.VMEM((16, H), jnp.float32),
                   pltpu.VMEM((16, H), jnp.float32),
                   pltpu.VMEM((TPW,), jnp.int32), pltpu.VMEM((TPW,), jnp.int32),
                   pltpu.VMEM_SHARED((2, S), jnp.int32),
                   pltpu.SemaphoreType.DMA, pltpu.SemaphoreType.DMA,
                   pltpu.SemaphoreType.DMA, pltpu.SemaphoreType.DMA,
                   pltpu.SemaphoreType.DMA],
    compiler_params=_SC_PARAMS,
)(_sort_body)


def _ffn_body(be_ref, xs_ref, w1_ref, w3_ref, w2_ref, ys_ref):
    del be_ref
    x = xs_ref[...].astype(jnp.bfloat16)
    h1 = lax.dot_general(x, w1_ref[0], (((1,), (1,)), ((), ())),
                         preferred_element_type=jnp.float32)
    h3 = lax.dot_general(x, w3_ref[0], (((1,), (1,)), ((), ())),
                         preferred_element_type=jnp.float32)
    h = (h1 * jax.nn.sigmoid(h1)) * h3
    y = lax.dot_general(h.astype(jnp.bfloat16), w2_ref[0],
                        (((1,), (1,)), ((), ())),
                        preferred_element_type=jnp.float32)
    ys_ref[...] = y


def _ffn(blk_exp, xs, w1b, w3b, w2b):
    return pl.pallas_call(
        _ffn_body,
        grid_spec=pltpu.PrefetchScalarGridSpec(
            num_scalar_prefetch=1,
            grid=(NB,),
            in_specs=[pl.BlockSpec((BM, H), lambda i, be: (i, 0)),
                      pl.BlockSpec((1, FF, H), lambda i, be: (be[i], 0, 0)),
                      pl.BlockSpec((1, FF, H), lambda i, be: (be[i], 0, 0)),
                      pl.BlockSpec((1, H, FF), lambda i, be: (be[i], 0, 0))],
            out_specs=pl.BlockSpec((BM, H), lambda i, be: (i, 0)),
        ),
        out_shape=jax.ShapeDtypeStruct((NP, H), jnp.float32),
        compiler_params=pltpu.CompilerParams(
            vmem_limit_bytes=63 * 1024 * 1024),
    )(blk_exp, xs, w1b, w3b, w2b)


def _combine_body(ys_hbm, sw_hbm, d0_hbm, d1_hbm, out_hbm,
                  i0v, i1v, w0v, w1v, ga, gb, gc,
                  sem0, sem1, sema, semc):
    cid = lax.axis_index("c")
    sid = lax.axis_index("s")
    wid = sid * 2 + cid
    base = wid * TPW
    pltpu.sync_copy(d0_hbm.at[pl.ds(base, TPW)], i0v)
    pltpu.sync_copy(d1_hbm.at[pl.ds(base, TPW)], i1v)
    pltpu.sync_copy(sw_hbm.at[0, pl.ds(base, TPW)], w0v)
    pltpu.sync_copy(sw_hbm.at[1, pl.ds(base, TPW)], w1v)
    stores = [None, None]
    for cch in range(TPW // 16):
        buf = ga if cch % 2 == 0 else gc
        stsem = sema if cch % 2 == 0 else semc
        if stores[cch % 2] is not None:
            stores[cch % 2].wait()
        iv0 = i0v[pl.ds(cch * 16, 16)]
        iv1 = i1v[pl.ds(cch * 16, 16)]
        cp0 = pltpu.async_copy(ys_hbm.at[iv0], buf, sem0)
        cp1 = pltpu.async_copy(ys_hbm.at[iv1], gb, sem1)
        cp0.wait()
        cp1.wait()
        wa = w0v[pl.ds(cch * 16, 16)]
        wb = w1v[pl.ds(cch * 16, 16)]

        @pl.loop(0, 16)
        def _(t):
            was = _splat(wa, t)
            wbs = _splat(wb, t)

            @pl.loop(0, 8)
            def _(u):
                for uu in range(16):
                    sl = pl.ds(u * 256 + uu * 16, 16)
                    buf[t, sl] = buf[t, sl] * was + gb[t, sl] * wbs

        stores[cch % 2] = pltpu.async_copy(
            buf, out_hbm.at[pl.ds(base + cch * 16, 16)], stsem)
    for st in stores:
        if st is not None:
            st.wait()


_combine_kernel = functools.partial(
    pl.kernel,
    out_type=jax.ShapeDtypeStruct((S, H), jnp.float32),
    mesh=_MESH,
    scratch_types=[pltpu.VMEM((TPW,), jnp.int32), pltpu.VMEM((TPW,), jnp.int32),
                   pltpu.VMEM((TPW,), jnp.float32),
                   pltpu.VMEM((TPW,), jnp.float32),
                   pltpu.VMEM((16, H), jnp.float32),
                   pltpu.VMEM((16, H), jnp.float32),
                   pltpu.VMEM((16, H), jnp.float32),
                   pltpu.SemaphoreType.DMA, pltpu.SemaphoreType.DMA,
                   pltpu.SemaphoreType.DMA, pltpu.SemaphoreType.DMA],
    compiler_params=_SC_PARAMS,
)(_combine_body)


def kernel(hidden_states, gate_w, w1, w3, w2):
    x = hidden_states.reshape(S, H)
    w1b = w1.astype(jnp.bfloat16)
    w3b = w3.astype(jnp.bfloat16)
    w2b = w2.astype(jnp.bfloat16)

    selw, seli = _router(x, gate_w)

    xs, d0, d1, blk_exp = _sort_kernel(x, seli)

    ys = _ffn(blk_exp, xs, w1b, w3b, w2b)

    out = _combine_kernel(ys, selw, d0, d1)
    return out.reshape(hidden_states.shape)
